# baseline (speedup 1.0000x reference)
import jax
import jax.numpy as jnp
from jax.experimental import pallas as pl
from jax.experimental.pallas import tpu as pltpu

H = 2048
E = 16
K = 2
CHUNK = 512


def _router_body(x_ref, w_ref, w1_ref, w2_ref, i1_ref, i2_ref, aux_ref,
                 cnt_acc, ps_acc):
    step = pl.program_id(0)
    nsteps = pl.num_programs(0)
    x = x_ref[...]
    logits = jnp.dot(x, w_ref[...], preferred_element_type=jnp.float32)

    m = jnp.max(logits, axis=1, keepdims=True)
    e = jnp.exp(logits - m)
    s = jnp.sum(e, axis=1, keepdims=True)
    p = e / s

    eidx = jax.lax.broadcasted_iota(jnp.int32, (CHUNK, E), 1)
    m1 = jnp.max(p, axis=1, keepdims=True)
    i1 = jnp.min(jnp.where(p == m1, eidx, E), axis=1, keepdims=True)
    p_m = jnp.where(eidx == i1, -jnp.inf, p)
    m2 = jnp.max(p_m, axis=1, keepdims=True)
    i2 = jnp.min(jnp.where(p_m == m2, eidx, E), axis=1, keepdims=True)

    tot = m1 + m2
    w1_ref[...] = (m1 / tot)[:, 0]
    w2_ref[...] = (m2 / tot)[:, 0]
    i1_ref[...] = i1[:, 0]
    i2_ref[...] = i2[:, 0]

    cnt = jnp.sum((eidx == i1).astype(jnp.float32)
                  + (eidx == i2).astype(jnp.float32), axis=0)
    ps = jnp.sum(p, axis=0)

    @pl.when(step == 0)
    def _init():
        cnt_acc[0, :] = cnt
        ps_acc[0, :] = ps

    @pl.when(step != 0)
    def _accum():
        cnt_acc[0, :] += cnt
        ps_acc[0, :] += ps

    @pl.when(step == nsteps - 1)
    def _finish():
        num_tokens = nsteps * CHUNK
        aux_ref[0, 0] = (jnp.sum(cnt_acc[0, :] * ps_acc[0, :])
                         * E / (num_tokens * num_tokens))


def kernel(hidden_states, W):
    B, S, _ = hidden_states.shape
    T = B * S
    x = hidden_states.reshape(T, H)
    grid = (T // CHUNK,)
    w1, w2, i1, i2, aux = pl.pallas_call(
        _router_body,
        grid=grid,
        in_specs=[
            pl.BlockSpec((CHUNK, H), lambda i: (i, 0)),
            pl.BlockSpec((H, E), lambda i: (0, 0)),
        ],
        out_specs=[
            pl.BlockSpec((CHUNK,), lambda i: (i,)),
            pl.BlockSpec((CHUNK,), lambda i: (i,)),
            pl.BlockSpec((CHUNK,), lambda i: (i,)),
            pl.BlockSpec((CHUNK,), lambda i: (i,)),
            pl.BlockSpec((1, 1), lambda i: (0, 0),
                         memory_space=pltpu.SMEM),
        ],
        out_shape=[
            jax.ShapeDtypeStruct((T,), jnp.float32),
            jax.ShapeDtypeStruct((T,), jnp.float32),
            jax.ShapeDtypeStruct((T,), jnp.int32),
            jax.ShapeDtypeStruct((T,), jnp.int32),
            jax.ShapeDtypeStruct((1, 1), jnp.float32),
        ],
        scratch_shapes=[
            pltpu.VMEM((1, E), jnp.float32),
            pltpu.VMEM((1, E), jnp.float32),
        ],
    )(x, W)
    top_k_weights = jnp.stack([w1, w2], axis=-1).reshape(B, S, K)
    top_k_indices = jnp.stack([i1, i2], axis=-1).reshape(B, S, K)
    return top_k_weights, top_k_indices, aux[0, 0]


# speedup vs baseline: 1.8330x; 1.8330x over previous
import jax
import jax.numpy as jnp
from jax.experimental import pallas as pl
from jax.experimental.pallas import tpu as pltpu

H = 2048
E = 16
K = 2
CHUNK = 512


def _router_body(x_ref, w_ref, w1_ref, w2_ref, i1_ref, i2_ref, aux_ref,
                 cnt_acc, ps_acc):
    step = pl.program_id(0)
    nsteps = pl.num_programs(0)
    x = x_ref[...]
    logits = jnp.dot(x, w_ref[...], preferred_element_type=jnp.float32)
    lt = logits.T

    m = jnp.max(lt, axis=0, keepdims=True)
    e = jnp.exp(lt - m)
    s = jnp.sum(e, axis=0, keepdims=True)
    p = e / s

    eidx = jax.lax.broadcasted_iota(jnp.int32, (E, CHUNK), 0)
    m1 = jnp.max(p, axis=0, keepdims=True)
    i1 = jnp.min(jnp.where(p == m1, eidx, E), axis=0, keepdims=True)
    hit1 = eidx == i1
    p_m = jnp.where(hit1, -jnp.inf, p)
    m2 = jnp.max(p_m, axis=0, keepdims=True)
    i2 = jnp.min(jnp.where(p_m == m2, eidx, E), axis=0, keepdims=True)
    hit2 = eidx == i2

    tot = m1 + m2
    w1_ref[...] = (m1 / tot).reshape(CHUNK)
    w2_ref[...] = (m2 / tot).reshape(CHUNK)
    i1_ref[...] = i1.reshape(CHUNK)
    i2_ref[...] = i2.reshape(CHUNK)

    cnt = jnp.sum(hit1.astype(jnp.float32) + hit2.astype(jnp.float32),
                  axis=1, keepdims=True)
    ps = jnp.sum(p, axis=1, keepdims=True)

    @pl.when(step == 0)
    def _init():
        cnt_acc[...] = cnt
        ps_acc[...] = ps

    @pl.when(step != 0)
    def _accum():
        cnt_acc[...] += cnt
        ps_acc[...] += ps

    @pl.when(step == nsteps - 1)
    def _finish():
        num_tokens = nsteps * CHUNK
        aux_ref[0, 0] = (jnp.sum(cnt_acc[...] * ps_acc[...])
                         * E / (num_tokens * num_tokens))


def kernel(hidden_states, W):
    B, S, _ = hidden_states.shape
    T = B * S
    x = hidden_states.reshape(T, H)
    grid = (T // CHUNK,)
    w1, w2, i1, i2, aux = pl.pallas_call(
        _router_body,
        grid=grid,
        in_specs=[
            pl.BlockSpec((CHUNK, H), lambda i: (i, 0)),
            pl.BlockSpec((H, E), lambda i: (0, 0)),
        ],
        out_specs=[
            pl.BlockSpec((CHUNK,), lambda i: (i,)),
            pl.BlockSpec((CHUNK,), lambda i: (i,)),
            pl.BlockSpec((CHUNK,), lambda i: (i,)),
            pl.BlockSpec((CHUNK,), lambda i: (i,)),
            pl.BlockSpec((1, 1), lambda i: (0, 0),
                         memory_space=pltpu.SMEM),
        ],
        out_shape=[
            jax.ShapeDtypeStruct((T,), jnp.float32),
            jax.ShapeDtypeStruct((T,), jnp.float32),
            jax.ShapeDtypeStruct((T,), jnp.int32),
            jax.ShapeDtypeStruct((T,), jnp.int32),
            jax.ShapeDtypeStruct((1, 1), jnp.float32),
        ],
        scratch_shapes=[
            pltpu.VMEM((E, 1), jnp.float32),
            pltpu.VMEM((E, 1), jnp.float32),
        ],
    )(x, W)
    top_k_weights = jnp.stack([w1, w2], axis=-1).reshape(B, S, K)
    top_k_indices = jnp.stack([i1, i2], axis=-1).reshape(B, S, K)
    return top_k_weights, top_k_indices, aux[0, 0]


# speedup vs baseline: 2.1700x; 1.1839x over previous
import jax
import jax.numpy as jnp
from jax.experimental import pallas as pl
from jax.experimental.pallas import tpu as pltpu

H = 2048
E = 16
K = 2
CHUNK = 2048


def _router_body(x_ref, w_ref, w1_ref, w2_ref, i1_ref, i2_ref, aux_ref,
                 cnt_acc, ps_acc):
    step = pl.program_id(0)
    nsteps = pl.num_programs(0)
    x = x_ref[...]
    logits = jnp.dot(x, w_ref[...], preferred_element_type=jnp.float32)
    lt = logits.T

    m = jnp.max(lt, axis=0, keepdims=True)
    e = jnp.exp(lt - m)
    s = jnp.sum(e, axis=0, keepdims=True)
    p = e / s

    eidx = jax.lax.broadcasted_iota(jnp.int32, (E, CHUNK), 0)
    m1 = jnp.max(p, axis=0, keepdims=True)
    i1 = jnp.min(jnp.where(p == m1, eidx, E), axis=0, keepdims=True)
    hit1 = eidx == i1
    p_m = jnp.where(hit1, -jnp.inf, p)
    m2 = jnp.max(p_m, axis=0, keepdims=True)
    i2 = jnp.min(jnp.where(p_m == m2, eidx, E), axis=0, keepdims=True)
    hit2 = eidx == i2

    tot = m1 + m2
    w1_ref[...] = (m1 / tot).reshape(CHUNK)
    w2_ref[...] = (m2 / tot).reshape(CHUNK)
    i1_ref[...] = i1.reshape(CHUNK)
    i2_ref[...] = i2.reshape(CHUNK)

    cnt = jnp.sum(hit1.astype(jnp.float32) + hit2.astype(jnp.float32),
                  axis=1, keepdims=True)
    ps = jnp.sum(p, axis=1, keepdims=True)

    @pl.when(step == 0)
    def _init():
        cnt_acc[...] = cnt
        ps_acc[...] = ps

    @pl.when(step != 0)
    def _accum():
        cnt_acc[...] += cnt
        ps_acc[...] += ps

    @pl.when(step == nsteps - 1)
    def _finish():
        num_tokens = nsteps * CHUNK
        aux_ref[0, 0] = (jnp.sum(cnt_acc[...] * ps_acc[...])
                         * E / (num_tokens * num_tokens))


def kernel(hidden_states, W):
    B, S, _ = hidden_states.shape
    T = B * S
    x = hidden_states.reshape(T, H)
    grid = (T // CHUNK,)
    w1, w2, i1, i2, aux = pl.pallas_call(
        _router_body,
        grid=grid,
        in_specs=[
            pl.BlockSpec((CHUNK, H), lambda i: (i, 0)),
            pl.BlockSpec((H, E), lambda i: (0, 0)),
        ],
        out_specs=[
            pl.BlockSpec((CHUNK,), lambda i: (i,)),
            pl.BlockSpec((CHUNK,), lambda i: (i,)),
            pl.BlockSpec((CHUNK,), lambda i: (i,)),
            pl.BlockSpec((CHUNK,), lambda i: (i,)),
            pl.BlockSpec((1, 1), lambda i: (0, 0),
                         memory_space=pltpu.SMEM),
        ],
        out_shape=[
            jax.ShapeDtypeStruct((T,), jnp.float32),
            jax.ShapeDtypeStruct((T,), jnp.float32),
            jax.ShapeDtypeStruct((T,), jnp.int32),
            jax.ShapeDtypeStruct((T,), jnp.int32),
            jax.ShapeDtypeStruct((1, 1), jnp.float32),
        ],
        scratch_shapes=[
            pltpu.VMEM((E, 1), jnp.float32),
            pltpu.VMEM((E, 1), jnp.float32),
        ],
    )(x, W)
    top_k_weights = jnp.stack([w1, w2], axis=-1).reshape(B, S, K)
    top_k_indices = jnp.stack([i1, i2], axis=-1).reshape(B, S, K)
    return top_k_weights, top_k_indices, aux[0, 0]


# speedup vs baseline: 2.1766x; 1.0030x over previous
import jax
import jax.numpy as jnp
from jax.experimental import pallas as pl
from jax.experimental.pallas import tpu as pltpu

H = 2048
E = 16
K = 2
CHUNK = 1024


def _router_body(x_ref, w_ref, w1_ref, w2_ref, i1_ref, i2_ref, aux_ref,
                 cnt_acc, ps_acc):
    step = pl.program_id(0)
    nsteps = pl.num_programs(0)
    x = x_ref[...]
    logits = jnp.dot(x, w_ref[...], preferred_element_type=jnp.float32)
    lt = logits.T

    m = jnp.max(lt, axis=0, keepdims=True)
    e = jnp.exp(lt - m)
    s = jnp.sum(e, axis=0, keepdims=True)
    p = e / s

    eidx = jax.lax.broadcasted_iota(jnp.int32, (E, CHUNK), 0)
    m1 = jnp.max(p, axis=0, keepdims=True)
    i1 = jnp.min(jnp.where(p == m1, eidx, E), axis=0, keepdims=True)
    hit1 = eidx == i1
    p_m = jnp.where(hit1, -jnp.inf, p)
    m2 = jnp.max(p_m, axis=0, keepdims=True)
    i2 = jnp.min(jnp.where(p_m == m2, eidx, E), axis=0, keepdims=True)
    hit2 = eidx == i2

    tot = m1 + m2
    w1_ref[...] = (m1 / tot).reshape(CHUNK)
    w2_ref[...] = (m2 / tot).reshape(CHUNK)
    i1_ref[...] = i1.reshape(CHUNK)
    i2_ref[...] = i2.reshape(CHUNK)

    cnt = jnp.sum(hit1.astype(jnp.float32) + hit2.astype(jnp.float32),
                  axis=1, keepdims=True)
    ps = jnp.sum(p, axis=1, keepdims=True)

    @pl.when(step == 0)
    def _init():
        cnt_acc[...] = cnt
        ps_acc[...] = ps

    @pl.when(step != 0)
    def _accum():
        cnt_acc[...] += cnt
        ps_acc[...] += ps

    @pl.when(step == nsteps - 1)
    def _finish():
        num_tokens = nsteps * CHUNK
        aux_ref[0, 0] = (jnp.sum(cnt_acc[...] * ps_acc[...])
                         * E / (num_tokens * num_tokens))


def kernel(hidden_states, W):
    B, S, _ = hidden_states.shape
    T = B * S
    x = hidden_states.reshape(T, H)
    grid = (T // CHUNK,)
    w1, w2, i1, i2, aux = pl.pallas_call(
        _router_body,
        grid=grid,
        in_specs=[
            pl.BlockSpec((CHUNK, H), lambda i: (i, 0)),
            pl.BlockSpec((H, E), lambda i: (0, 0)),
        ],
        out_specs=[
            pl.BlockSpec((CHUNK,), lambda i: (i,)),
            pl.BlockSpec((CHUNK,), lambda i: (i,)),
            pl.BlockSpec((CHUNK,), lambda i: (i,)),
            pl.BlockSpec((CHUNK,), lambda i: (i,)),
            pl.BlockSpec((1, 1), lambda i: (0, 0),
                         memory_space=pltpu.SMEM),
        ],
        out_shape=[
            jax.ShapeDtypeStruct((T,), jnp.float32),
            jax.ShapeDtypeStruct((T,), jnp.float32),
            jax.ShapeDtypeStruct((T,), jnp.int32),
            jax.ShapeDtypeStruct((T,), jnp.int32),
            jax.ShapeDtypeStruct((1, 1), jnp.float32),
        ],
        scratch_shapes=[
            pltpu.VMEM((E, 1), jnp.float32),
            pltpu.VMEM((E, 1), jnp.float32),
        ],
    )(x, W)
    top_k_weights = jnp.stack([w1, w2], axis=-1).reshape(B, S, K)
    top_k_indices = jnp.stack([i1, i2], axis=-1).reshape(B, S, K)
    return top_k_weights, top_k_indices, aux[0, 0]


# speedup vs baseline: 2.1831x; 1.0030x over previous
import jax
import jax.numpy as jnp
from jax.experimental import pallas as pl
from jax.experimental.pallas import tpu as pltpu

H = 2048
E = 16
K = 2
CHUNK = 1024


def _router_body(xa_ref, xb_ref, w_ref, w1_ref, w2_ref, i1_ref, i2_ref,
                 aux_ref, cnt_acc, ps_acc):
    step = pl.program_id(0)
    nsteps = pl.num_programs(0)
    logits = (jnp.dot(xa_ref[...], w_ref[:H // 2, :],
                      preferred_element_type=jnp.float32)
              + jnp.dot(xb_ref[...], w_ref[H // 2:, :],
                        preferred_element_type=jnp.float32))
    lt = logits.T

    m = jnp.max(lt, axis=0, keepdims=True)
    e = jnp.exp(lt - m)
    s = jnp.sum(e, axis=0, keepdims=True)
    p = e / s

    eidx = jax.lax.broadcasted_iota(jnp.int32, (E, CHUNK), 0)
    m1 = jnp.max(p, axis=0, keepdims=True)
    i1 = jnp.min(jnp.where(p == m1, eidx, E), axis=0, keepdims=True)
    hit1 = eidx == i1
    p_m = jnp.where(hit1, -jnp.inf, p)
    m2 = jnp.max(p_m, axis=0, keepdims=True)
    i2 = jnp.min(jnp.where(p_m == m2, eidx, E), axis=0, keepdims=True)
    hit2 = eidx == i2

    tot = m1 + m2
    w1_ref[...] = (m1 / tot).reshape(CHUNK)
    w2_ref[...] = (m2 / tot).reshape(CHUNK)
    i1_ref[...] = i1.reshape(CHUNK)
    i2_ref[...] = i2.reshape(CHUNK)

    cnt = jnp.sum(hit1.astype(jnp.float32) + hit2.astype(jnp.float32),
                  axis=1, keepdims=True)
    ps = jnp.sum(p, axis=1, keepdims=True)

    @pl.when(step == 0)
    def _init():
        cnt_acc[...] = cnt
        ps_acc[...] = ps

    @pl.when(step != 0)
    def _accum():
        cnt_acc[...] += cnt
        ps_acc[...] += ps

    @pl.when(step == nsteps - 1)
    def _finish():
        num_tokens = nsteps * CHUNK
        aux_ref[0, 0] = (jnp.sum(cnt_acc[...] * ps_acc[...])
                         * E / (num_tokens * num_tokens))


def kernel(hidden_states, W):
    B, S, _ = hidden_states.shape
    T = B * S
    x = hidden_states.reshape(T, H)
    grid = (T // CHUNK,)
    w1, w2, i1, i2, aux = pl.pallas_call(
        _router_body,
        grid=grid,
        in_specs=[
            pl.BlockSpec((CHUNK, H // 2), lambda i: (i, 0)),
            pl.BlockSpec((CHUNK, H // 2), lambda i: (i, 1)),
            pl.BlockSpec((H, E), lambda i: (0, 0)),
        ],
        out_specs=[
            pl.BlockSpec((CHUNK,), lambda i: (i,)),
            pl.BlockSpec((CHUNK,), lambda i: (i,)),
            pl.BlockSpec((CHUNK,), lambda i: (i,)),
            pl.BlockSpec((CHUNK,), lambda i: (i,)),
            pl.BlockSpec((1, 1), lambda i: (0, 0),
                         memory_space=pltpu.SMEM),
        ],
        out_shape=[
            jax.ShapeDtypeStruct((T,), jnp.float32),
            jax.ShapeDtypeStruct((T,), jnp.float32),
            jax.ShapeDtypeStruct((T,), jnp.int32),
            jax.ShapeDtypeStruct((T,), jnp.int32),
            jax.ShapeDtypeStruct((1, 1), jnp.float32),
        ],
        scratch_shapes=[
            pltpu.VMEM((E, 1), jnp.float32),
            pltpu.VMEM((E, 1), jnp.float32),
        ],
    )(x, x, W)
    top_k_weights = jnp.stack([w1, w2], axis=-1).reshape(B, S, K)
    top_k_indices = jnp.stack([i1, i2], axis=-1).reshape(B, S, K)
    return top_k_weights, top_k_indices, aux[0, 0]
